# baseline (speedup 1.0000x reference)
import functools

import numpy as np
import jax
import jax.numpy as jnp
from jax import lax
from jax.experimental import pallas as pl
from jax.experimental.pallas import tpu as pltpu
from jax.experimental.pallas import tpu_sc as plsc

_N = 64
_M = 128
_NT = 16384
_NC = 2
_PPW = _NT // 16
_TAU = 3.0 / (np.pi * np.sqrt(128.0))
_WCOEF = -1.0 / (16.0 * _TAU)


def _constants():
    x = np.arange(_N)
    rx = (x - _N // 2) / float(_N)
    psi_hat = np.sqrt(4.0 * np.pi * _TAU) * np.exp(-4.0 * np.pi**2 * _TAU * rx**2)
    ax = 0.5 / psi_hat
    a2d = np.outer(ax, ax).astype(np.float32)
    m = np.arange(_M)
    ang = -2.0 * np.pi * np.outer(m - _M // 2, x - _N // 2) / float(_M)
    dr = np.cos(ang)
    di = np.sin(ang)
    dts = np.concatenate([dr.T, di.T], axis=0).astype(np.float32)
    d1s = np.block([[dr, -di], [di, dr]]).astype(np.float32)
    return a2d, dts, d1s


_A2D, _DTS, _D1S = _constants()


def _tc_grid_body(ir_ref, ii_ref, a_ref, dts_ref, d1s_ref, out_ref):
    a = a_ref[...]
    for c in range(_NC):
        br = ir_ref[c] * a
        bi = ii_ref[c] * a
        bs = jnp.concatenate(
            [
                jnp.concatenate([br, -bi], axis=1),
                jnp.concatenate([bi, br], axis=1),
            ],
            axis=0,
        )
        p = jnp.dot(bs, dts_ref[...], preferred_element_type=jnp.float32,
                    precision=jax.lax.Precision.HIGHEST)
        gc = jnp.dot(d1s_ref[...], p, preferred_element_type=jnp.float32,
                     precision=jax.lax.Precision.HIGHEST)
        out_ref[c] = gc


def _make_grid(img_real, img_imag):
    return pl.pallas_call(
        _tc_grid_body,
        out_shape=jax.ShapeDtypeStruct((_NC, 2 * _M, _M), jnp.float32),
    )(img_real[0], img_imag[0], _A2D, _DTS, _D1S)


_SC_MESH = plsc.VectorSubcoreMesh(core_axis_name="c", subcore_axis_name="s")


@functools.partial(
    pl.kernel,
    mesh=_SC_MESH,
    out_type=[
        jax.ShapeDtypeStruct((_NC, _NT), jnp.float32),
        jax.ShapeDtypeStruct((_NC, _NT), jnp.float32),
    ],
    scratch_types=[
        pltpu.VMEM((_M * _M,), jnp.float32),
        pltpu.VMEM((_M * _M,), jnp.float32),
        pltpu.VMEM((_PPW,), jnp.float32),
        pltpu.VMEM((_PPW,), jnp.float32),
        pltpu.VMEM((_PPW,), jnp.float32),
        pltpu.VMEM((_PPW,), jnp.float32),
    ],
    compiler_params=pltpu.CompilerParams(needs_layout_passes=False),
)
def _sc_interp(grid_hbm, tx_hbm, ty_hbm, outr_hbm, outi_hbm,
               grr, gri, txv, tyv, orv, oiv):
    cid = lax.axis_index("c")
    sid = lax.axis_index("s")
    base = sid * _PPW
    pltpu.sync_copy(grid_hbm.at[cid, 0], grr)
    pltpu.sync_copy(grid_hbm.at[cid, 1], gri)
    pltpu.sync_copy(tx_hbm.at[pl.ds(base, _PPW)], txv)
    pltpu.sync_copy(ty_hbm.at[pl.ds(base, _PPW)], tyv)

    def body(gidx, carry):
        o = gidx * 16
        gx = txv[pl.ds(o, 16)] * float(_M)
        gy = tyv[pl.ds(o, 16)] * float(_M)
        ix = gx.astype(jnp.int32)
        iy = gy.astype(jnp.int32)
        fx = gx - ix.astype(jnp.float32)
        fy = gy - iy.astype(jnp.float32)
        wx = [jnp.exp(((fx - j) * (fx - j)) * _WCOEF) for j in range(-2, 4)]
        wy = [jnp.exp(((fy - l) * (fy - l)) * _WCOEF) for l in range(-2, 4)]
        rows = [((ix + j) & (_M - 1)) * _M for j in range(-2, 4)]
        cols = [(iy + l) & (_M - 1) for l in range(-2, 4)]
        accr = jnp.zeros((16,), jnp.float32)
        acci = jnp.zeros((16,), jnp.float32)
        for j in range(6):
            for l in range(6):
                idx = rows[j] + cols[l]
                w = wx[j] * wy[l]
                accr = accr + w * plsc.load_gather(grr, [idx])
                acci = acci + w * plsc.load_gather(gri, [idx])
        orv[pl.ds(o, 16)] = accr
        oiv[pl.ds(o, 16)] = acci
        return carry

    lax.fori_loop(0, _PPW // 16, body, 0)
    pltpu.sync_copy(orv, outr_hbm.at[cid, pl.ds(base, _PPW)])
    pltpu.sync_copy(oiv, outi_hbm.at[cid, pl.ds(base, _PPW)])


@jax.jit
def _nufft(img_real, img_imag, trj):
    grid = _make_grid(img_real, img_imag)
    grid_planes = grid.reshape(_NC, 2, _M * _M)
    tx = trj[0, :, 0]
    ty = trj[0, :, 1]
    out_r, out_i = _sc_interp(grid_planes, tx, ty)
    return (out_r + 1j * out_i).astype(jnp.complex64)[None]


def kernel(img_real, img_imag, trj):
    return _nufft(img_real, img_imag, trj)


# speedup vs baseline: 1.0074x; 1.0074x over previous
import functools

import numpy as np
import jax
import jax.numpy as jnp
from jax import lax
from jax.experimental import pallas as pl
from jax.experimental.pallas import tpu as pltpu
from jax.experimental.pallas import tpu_sc as plsc

_N = 64
_M = 128
_NT = 16384
_NC = 2
_PPW = _NT // 16
_TAU = 3.0 / (np.pi * np.sqrt(128.0))
_WCOEF = -1.0 / (16.0 * _TAU)


def _constants():
    x = np.arange(_N)
    rx = (x - _N // 2) / float(_N)
    psi_hat = np.sqrt(4.0 * np.pi * _TAU) * np.exp(-4.0 * np.pi**2 * _TAU * rx**2)
    ax = 0.5 / psi_hat
    a2d = np.outer(ax, ax).astype(np.float32)
    m = np.arange(_M)
    ang = -2.0 * np.pi * np.outer(m - _M // 2, x - _N // 2) / float(_M)
    dr = np.cos(ang)
    di = np.sin(ang)
    dts = np.concatenate([dr.T, di.T], axis=0).astype(np.float32)
    d1s = np.block([[dr, -di], [di, dr]]).astype(np.float32)
    return a2d, dts, d1s


_A2D, _DTS, _D1S = _constants()


def _tc_grid_body(ir_ref, ii_ref, a_ref, dts_ref, d1s_ref, out_ref):
    a = a_ref[...]
    for c in range(_NC):
        br = ir_ref[c] * a
        bi = ii_ref[c] * a
        bs = jnp.concatenate(
            [
                jnp.concatenate([br, -bi], axis=1),
                jnp.concatenate([bi, br], axis=1),
            ],
            axis=0,
        )
        p = jnp.dot(bs, dts_ref[...], preferred_element_type=jnp.float32,
                    precision=jax.lax.Precision.HIGHEST)
        gc = jnp.dot(d1s_ref[...], p, preferred_element_type=jnp.float32,
                     precision=jax.lax.Precision.HIGHEST)
        out_ref[c] = gc


def _make_grid(img_real, img_imag):
    return pl.pallas_call(
        _tc_grid_body,
        out_shape=jax.ShapeDtypeStruct((_NC, 2 * _M, _M), jnp.float32),
    )(img_real[0], img_imag[0], _A2D, _DTS, _D1S)


_SC_MESH = plsc.VectorSubcoreMesh(core_axis_name="c", subcore_axis_name="s")


@functools.partial(
    pl.kernel,
    mesh=_SC_MESH,
    out_type=[
        jax.ShapeDtypeStruct((_NC, _NT), jnp.float32),
        jax.ShapeDtypeStruct((_NC, _NT), jnp.float32),
    ],
    scratch_types=[
        pltpu.VMEM((_M * _M,), jnp.float32),
        pltpu.VMEM((_M * _M,), jnp.float32),
        pltpu.VMEM((_PPW,), jnp.float32),
        pltpu.VMEM((_PPW,), jnp.float32),
        pltpu.VMEM((_PPW,), jnp.float32),
        pltpu.VMEM((_PPW,), jnp.float32),
    ],
    compiler_params=pltpu.CompilerParams(needs_layout_passes=False),
)
def _sc_interp(grid_hbm, tx_hbm, ty_hbm, outr_hbm, outi_hbm,
               grr, gri, txv, tyv, orv, oiv):
    cid = lax.axis_index("c")
    sid = lax.axis_index("s")
    base = sid * _PPW
    pltpu.sync_copy(grid_hbm.at[cid, 0], grr)
    pltpu.sync_copy(grid_hbm.at[cid, 1], gri)
    pltpu.sync_copy(tx_hbm.at[pl.ds(base, _PPW)], txv)
    pltpu.sync_copy(ty_hbm.at[pl.ds(base, _PPW)], tyv)

    def body(gidx, carry):
        o = gidx * 16
        gx = txv[pl.ds(o, 16)] * float(_M)
        gy = tyv[pl.ds(o, 16)] * float(_M)
        ix = gx.astype(jnp.int32)
        iy = gy.astype(jnp.int32)
        fx = gx - ix.astype(jnp.float32)
        fy = gy - iy.astype(jnp.float32)
        wx = [jnp.exp(((fx - j) * (fx - j)) * _WCOEF) for j in range(-2, 4)]
        wy = [jnp.exp(((fy - l) * (fy - l)) * _WCOEF) for l in range(-2, 4)]
        rows = [((ix + j) & (_M - 1)) * _M for j in range(-2, 4)]
        cols = [(iy + l) & (_M - 1) for l in range(-2, 4)]
        rparts = []
        iparts = []
        for j in range(6):
            idx0 = rows[j] + cols[0]
            ar = wy[0] * plsc.load_gather(grr, [idx0])
            ai = wy[0] * plsc.load_gather(gri, [idx0])
            for l in range(1, 6):
                idx = rows[j] + cols[l]
                ar = ar + wy[l] * plsc.load_gather(grr, [idx])
                ai = ai + wy[l] * plsc.load_gather(gri, [idx])
            rparts.append(wx[j] * ar)
            iparts.append(wx[j] * ai)
        accr = ((rparts[0] + rparts[1]) + (rparts[2] + rparts[3])) + (
            rparts[4] + rparts[5])
        acci = ((iparts[0] + iparts[1]) + (iparts[2] + iparts[3])) + (
            iparts[4] + iparts[5])
        orv[pl.ds(o, 16)] = accr
        oiv[pl.ds(o, 16)] = acci
        return carry

    lax.fori_loop(0, _PPW // 16, body, 0)
    pltpu.sync_copy(orv, outr_hbm.at[cid, pl.ds(base, _PPW)])
    pltpu.sync_copy(oiv, outi_hbm.at[cid, pl.ds(base, _PPW)])


@jax.jit
def _nufft(img_real, img_imag, trj):
    grid = _make_grid(img_real, img_imag)
    grid_planes = grid.reshape(_NC, 2, _M * _M)
    tx = trj[0, :, 0]
    ty = trj[0, :, 1]
    out_r, out_i = _sc_interp(grid_planes, tx, ty)
    return (out_r + 1j * out_i).astype(jnp.complex64)[None]


def kernel(img_real, img_imag, trj):
    return _nufft(img_real, img_imag, trj)


# speedup vs baseline: 2.6857x; 2.6661x over previous
import functools

import jax
import jax.numpy as jnp
from jax.experimental import pallas as pl

_IM = 64
_NC = 2
_NT = 16384
_TB = 2048


def _build_exp(theta):
    wr = jnp.cos(theta)
    wi = jnp.sin(theta)
    pows = [(wr, wi)]
    for _ in range(5):
        pr, pi_ = pows[-1]
        pows.append((pr * pr - pi_ * pi_, 2.0 * pr * pi_))
    p32r, p32i = pows[5]
    er, ei = p32r, -p32i
    for s in range(6):
        pr, pi_ = pows[s]
        nr = er * pr - ei * pi_
        ni = er * pi_ + ei * pr
        er = jnp.concatenate([er, nr], axis=0)
        ei = jnp.concatenate([ei, ni], axis=0)
    return er, ei


def _body(trj_ref, a_ref, out_r_ref, out_i_ref):
    two_pi = 2.0 * jnp.pi
    th0 = (-two_pi) * (trj_ref[0:1, :] - 0.5)
    th1 = (-two_pi) * (trj_ref[1:2, :] - 0.5)
    e1r, e1i = _build_exp(th1)
    e1 = jnp.concatenate([e1r, e1i], axis=0)
    tmp = jnp.dot(a_ref[...], e1, preferred_element_type=jnp.float32)

    wr = jnp.cos(th0)
    wi = jnp.sin(th0)
    pows = [(wr, wi)]
    for _ in range(5):
        pr, pi_ = pows[-1]
        pows.append((pr * pr - pi_ * pi_, 2.0 * pr * pi_))
    er = jnp.concatenate([jnp.ones_like(wr), wr], axis=0)
    ei = jnp.concatenate([jnp.zeros_like(wi), wi], axis=0)
    for s in (1, 2):
        pr, pi_ = pows[s]
        er, ei = (
            jnp.concatenate([er, er * pr - ei * pi_], axis=0),
            jnp.concatenate([ei, er * pi_ + ei * pr], axis=0),
        )
    p32r, p32i = pows[5]

    outs_r = []
    outs_i = []
    for c in range(_NC):
        ar = tmp[128 * c:128 * c + 64]
        ai = tmp[128 * c + 64:128 * c + 128]
        for s in (5, 4, 3):
            pr, pi_ = pows[s]
            half = 1 << s
            lo_r, hi_r = ar[:half], ar[half:]
            lo_i, hi_i = ai[:half], ai[half:]
            ar = lo_r + (pr * hi_r - pi_ * hi_i)
            ai = lo_i + (pr * hi_i + pi_ * hi_r)
        sr = jnp.sum(er * ar - ei * ai, axis=0, keepdims=True)
        si = jnp.sum(er * ai + ei * ar, axis=0, keepdims=True)
        outs_r.append(sr * p32r + si * p32i)
        outs_i.append(si * p32r - sr * p32i)
    out_r_ref[...] = jnp.concatenate(outs_r, axis=0)
    out_i_ref[...] = jnp.concatenate(outs_i, axis=0)


@functools.partial(jax.jit, static_argnames=("interpret",))
def _nufft(img_real, img_imag, trj, interpret=False):
    ir = img_real[0]
    ii = img_imag[0]

    def coil_block(c):
        return jnp.concatenate(
            [
                jnp.concatenate([ir[c], -ii[c]], axis=1),
                jnp.concatenate([ii[c], ir[c]], axis=1),
            ],
            axis=0,
        )

    a = jnp.concatenate([coil_block(0), coil_block(1)], axis=0)
    trj_t = trj[0].T

    grid = (_NT // _TB,)
    out_r, out_i = pl.pallas_call(
        _body,
        grid=grid,
        in_specs=[
            pl.BlockSpec((2, _TB), lambda i: (0, i)),
            pl.BlockSpec((256, 128), lambda i: (0, 0)),
        ],
        out_specs=[
            pl.BlockSpec((_NC, _TB), lambda i: (0, i)),
            pl.BlockSpec((_NC, _TB), lambda i: (0, i)),
        ],
        out_shape=[
            jax.ShapeDtypeStruct((_NC, _NT), jnp.float32),
            jax.ShapeDtypeStruct((_NC, _NT), jnp.float32),
        ],
        interpret=interpret,
    )(trj_t, a)
    return (out_r + 1j * out_i).astype(jnp.complex64)[None]


def kernel(img_real, img_imag, trj):
    return _nufft(img_real, img_imag, trj)


# speedup vs baseline: 2.6921x; 1.0024x over previous
import functools

import jax
import jax.numpy as jnp
from jax.experimental import pallas as pl

_IM = 64
_NC = 2
_NT = 16384
_TB = 4096


def _build_exp(theta):
    wr = jnp.cos(theta)
    wi = jnp.sin(theta)
    pows = [(wr, wi)]
    for _ in range(5):
        pr, pi_ = pows[-1]
        pows.append((pr * pr - pi_ * pi_, 2.0 * pr * pi_))
    p32r, p32i = pows[5]
    er, ei = p32r, -p32i
    for s in range(6):
        pr, pi_ = pows[s]
        nr = er * pr - ei * pi_
        ni = er * pi_ + ei * pr
        er = jnp.concatenate([er, nr], axis=0)
        ei = jnp.concatenate([ei, ni], axis=0)
    return er, ei


def _body(trj_ref, a_ref, out_r_ref, out_i_ref):
    two_pi = 2.0 * jnp.pi
    th0 = (-two_pi) * (trj_ref[0:1, :] - 0.5)
    th1 = (-two_pi) * (trj_ref[1:2, :] - 0.5)
    e1r, e1i = _build_exp(th1)
    e1 = jnp.concatenate([e1r, e1i], axis=0)
    tmp = jnp.dot(a_ref[...], e1, preferred_element_type=jnp.float32)

    wr = jnp.cos(th0)
    wi = jnp.sin(th0)
    pows = [(wr, wi)]
    for _ in range(5):
        pr, pi_ = pows[-1]
        pows.append((pr * pr - pi_ * pi_, 2.0 * pr * pi_))
    er = jnp.concatenate([jnp.ones_like(wr), wr], axis=0)
    ei = jnp.concatenate([jnp.zeros_like(wi), wi], axis=0)
    for s in (1, 2):
        pr, pi_ = pows[s]
        er, ei = (
            jnp.concatenate([er, er * pr - ei * pi_], axis=0),
            jnp.concatenate([ei, er * pi_ + ei * pr], axis=0),
        )
    p32r, p32i = pows[5]

    outs_r = []
    outs_i = []
    for c in range(_NC):
        ar = tmp[128 * c:128 * c + 64]
        ai = tmp[128 * c + 64:128 * c + 128]
        for s in (5, 4, 3):
            pr, pi_ = pows[s]
            half = 1 << s
            lo_r, hi_r = ar[:half], ar[half:]
            lo_i, hi_i = ai[:half], ai[half:]
            ar = lo_r + (pr * hi_r - pi_ * hi_i)
            ai = lo_i + (pr * hi_i + pi_ * hi_r)
        sr = jnp.sum(er * ar - ei * ai, axis=0, keepdims=True)
        si = jnp.sum(er * ai + ei * ar, axis=0, keepdims=True)
        outs_r.append(sr * p32r + si * p32i)
        outs_i.append(si * p32r - sr * p32i)
    out_r_ref[...] = jnp.concatenate(outs_r, axis=0)
    out_i_ref[...] = jnp.concatenate(outs_i, axis=0)


@functools.partial(jax.jit, static_argnames=("interpret",))
def _nufft(img_real, img_imag, trj, interpret=False):
    ir = img_real[0]
    ii = img_imag[0]

    def coil_block(c):
        return jnp.concatenate(
            [
                jnp.concatenate([ir[c], -ii[c]], axis=1),
                jnp.concatenate([ii[c], ir[c]], axis=1),
            ],
            axis=0,
        )

    a = jnp.concatenate([coil_block(0), coil_block(1)], axis=0)
    trj_t = trj[0].T

    grid = (_NT // _TB,)
    out_r, out_i = pl.pallas_call(
        _body,
        grid=grid,
        in_specs=[
            pl.BlockSpec((2, _TB), lambda i: (0, i)),
            pl.BlockSpec((256, 128), lambda i: (0, 0)),
        ],
        out_specs=[
            pl.BlockSpec((_NC, _TB), lambda i: (0, i)),
            pl.BlockSpec((_NC, _TB), lambda i: (0, i)),
        ],
        out_shape=[
            jax.ShapeDtypeStruct((_NC, _NT), jnp.float32),
            jax.ShapeDtypeStruct((_NC, _NT), jnp.float32),
        ],
        interpret=interpret,
    )(trj_t, a)
    return (out_r + 1j * out_i).astype(jnp.complex64)[None]


def kernel(img_real, img_imag, trj):
    return _nufft(img_real, img_imag, trj)
